# Initial kernel scaffold; baseline (speedup 1.0000x reference)
import jax
import jax.numpy as jnp
from jax.experimental import pallas as pl


def kernel(x1, edge_index1, batch1, x2, edge_index2, batch2, Wc1, bc1, Wc2, bc2, Wl1, bl1, Wl2, bl2):
    raise NotImplementedError("write your pallas kernel here")


# speedup vs baseline: 7.4240x; 7.4240x over previous
import functools

import jax
import jax.numpy as jnp
from jax import lax
from jax.experimental import pallas as pl
from jax.experimental.pallas import tpu as pltpu
from jax.experimental.pallas import tpu_sc as plsc

G = 128
C_CH = 12800
K = 128
SA = 2048
NC = 2
NS = 16


def _ceil_to(x, m):
    return ((x + m - 1) // m) * m


def _vgather(x, idx):
    dnums = lax.GatherDimensionNumbers(
        offset_dims=(), collapsed_slice_dims=(0,), start_index_map=(0,))
    return lax.gather(x, idx[:, None], dnums, (1,),
                      mode=lax.GatherScatterMode.PROMISE_IN_BOUNDS)


def _make_deg(eall, hsz):
    ept = eall // (NC * NS)
    nstrips = ept // SA
    mesh = plsc.VectorSubcoreMesh(core_axis_name="c", subcore_axis_name="s")

    def body(dstall, part, hist, dstage):
        c = lax.axis_index("c")
        s = lax.axis_index("s")
        wid = s * NC + c
        z16 = jnp.zeros((16,), jnp.float32)

        def zb(i, _):
            hist[pl.ds(i * 16, 16)] = z16
            return 0

        lax.fori_loop(0, hsz // 16, zb, 0)
        io = lax.iota(jnp.int32, 16)

        def strip(st, _):
            pltpu.sync_copy(dstall.at[pl.ds(wid * ept + st * SA, SA)], dstage)

            def bb(b, _):
                d = dstage[pl.ds(b * 16, 16)]
                sd = jnp.sort(d)
                prev = _vgather(sd, jnp.maximum(io - 1, 0))
                nxt = _vgather(sd, jnp.minimum(io + 1, 15))
                first = (io == 0) | (sd != prev)
                last = (io == 15) | (sd != nxt)
                startpos = plsc.cummax(jnp.where(first, io, 0))
                cnt = (io + 1 - startpos).astype(jnp.float32)
                plsc.addupdate_scatter(hist, [sd], cnt, mask=last)
                return 0

            lax.fori_loop(0, SA // 16, bb, 0)
            return 0

        lax.fori_loop(0, nstrips, strip, 0)
        pltpu.sync_copy(hist, part.at[wid])

    return pl.kernel(
        body,
        out_type=jax.ShapeDtypeStruct((NC * NS, hsz), jnp.float32),
        mesh=mesh,
        compiler_params=pltpu.CompilerParams(needs_layout_passes=False),
        scratch_types=[
            pltpu.VMEM((hsz,), jnp.float32),
            pltpu.VMEM((SA,), jnp.int32),
        ],
    )


def _make_dinv(hsz, nrows):
    rows = nrows // (NC * NS)
    mesh = plsc.VectorSubcoreMesh(core_axis_name="c", subcore_axis_name="s")

    def body(part, dinv, acc, tbuf):
        c = lax.axis_index("c")
        s = lax.axis_index("s")
        wid = s * NC + c
        base = wid * rows
        pltpu.sync_copy(part.at[0, pl.ds(base, rows)], acc)
        for t in range(1, NC * NS):
            pltpu.sync_copy(part.at[t, pl.ds(base, rows)], tbuf)

            def ab(i, _):
                sl = pl.ds(i * 16, 16)
                acc[sl] = acc[sl] + tbuf[sl]
                return 0

            lax.fori_loop(0, rows // 16, ab, 0)

        def db(i, _):
            sl = pl.ds(i * 16, 16)
            deg = acc[sl] + 1.0
            ih = jnp.int32(0x5F3759DF) - (plsc.bitcast(deg, jnp.int32) >> 1)
            y = plsc.bitcast(ih, jnp.float32)
            y = y * (1.5 - 0.5 * deg * y * y)
            y = y * (1.5 - 0.5 * deg * y * y)
            y = y * (1.5 - 0.5 * deg * y * y)
            acc[sl] = y
            return 0

        lax.fori_loop(0, rows // 16, db, 0)
        pltpu.sync_copy(acc, dinv.at[pl.ds(base, rows)])

    return pl.kernel(
        body,
        out_type=jax.ShapeDtypeStruct((nrows,), jnp.float32),
        mesh=mesh,
        compiler_params=pltpu.CompilerParams(needs_layout_passes=False),
        scratch_types=[
            pltpu.VMEM((rows,), jnp.float32),
            pltpu.VMEM((rows,), jnp.float32),
        ],
    )


def _make_agg(eall, h, npass, nrows_out):
    etc = eall // NS
    nstrips = etc // SA
    rows_pt = C_CH // NS
    mesh = plsc.VectorSubcoreMesh(core_axis_name="c", subcore_axis_name="s")

    def body(g, srcall, dstall, out, acc, dstage, sstage, srcbuf,
             offbuf, stage, sem):
        c = lax.axis_index("c")
        s = lax.axis_index("s")
        io = lax.iota(jnp.int32, 16)
        zv = jnp.zeros((16,), jnp.float32)

        def ploop(p, _):
            lo = (NC * p + c) * C_CH

            def zb(i, _):
                stage[i >> 3, pl.ds((i & 7) * 16, 16)] = zv
                return 0

            lax.fori_loop(0, K * (h // 16), zb, 0)
            nfull = rows_pt // K
            for j in range(nfull):
                pltpu.sync_copy(stage, acc.at[pl.ds(s * rows_pt + j * K, K)])
            tail = rows_pt - nfull * K
            if tail:
                pltpu.sync_copy(stage.at[pl.ds(0, tail)],
                                acc.at[pl.ds(s * rows_pt + nfull * K, tail)])

            @pl.when(s == 0)
            def _():
                pltpu.sync_copy(stage.at[pl.ds(0, 16)], acc.at[pl.ds(C_CH, 16)])

            plsc.subcore_barrier()

            def strip(st, _):
                ebase = s * etc + st * SA
                pltpu.sync_copy(dstall.at[pl.ds(ebase, SA)], dstage)
                pltpu.sync_copy(srcall.at[pl.ds(ebase, SA)], sstage)

                def sb(b, nacc):
                    d = dstage[pl.ds(b * 16, 16)]
                    sv = sstage[pl.ds(b * 16, 16)]
                    m = (d >= lo) & (d < lo + C_CH)
                    mi = m.astype(jnp.int32)
                    cs = plsc.cumsum(mi)
                    pos = nacc + cs - 1
                    plsc.store_scatter(srcbuf, [pos], sv, mask=m)
                    plsc.store_scatter(offbuf, [pos], d - lo, mask=m)
                    return nacc + jnp.sum(mi)

                nmatch = lax.fori_loop(0, SA // 16, sb, jnp.int32(0))
                nb = (nmatch + (K - 1)) // K
                safe = io * 64 + (s * NC + c) * 16
                for j in range(K // 16):
                    pp = nmatch + j * 16 + io
                    pm = pp < nb * K
                    plsc.store_scatter(offbuf, [pp], C_CH + io, mask=pm)
                    plsc.store_scatter(srcbuf, [pp], safe, mask=pm)

                def drain(j, _):
                    cp = pltpu.async_copy(
                        g.at[srcbuf.at[pl.ds(j * K, K)]], stage, sem)
                    cp.wait()
                    pltpu.sync_copy(stage,
                                    acc.at[offbuf.at[pl.ds(j * K, K)]],
                                    add=True)
                    return 0

                lax.fori_loop(0, nb, drain, 0)
                return 0

            lax.fori_loop(0, nstrips, strip, 0)

            plsc.subcore_barrier()
            pltpu.sync_copy(acc.at[pl.ds(s * rows_pt, rows_pt)],
                            out.at[pl.ds(lo + s * rows_pt, rows_pt)])
            plsc.subcore_barrier()
            return 0

        lax.fori_loop(0, npass, ploop, 0)

    return pl.kernel(
        body,
        out_type=jax.ShapeDtypeStruct((nrows_out, h), jnp.float32),
        mesh=mesh,
        compiler_params=pltpu.CompilerParams(needs_layout_passes=False),
        scratch_types=[
            pltpu.VMEM_SHARED((C_CH + 16, h), jnp.float32),
            pltpu.VMEM((SA,), jnp.int32),
            pltpu.VMEM((SA,), jnp.int32),
            pltpu.VMEM((SA,), jnp.int32),
            pltpu.VMEM((SA,), jnp.int32),
            pltpu.VMEM((K, h), jnp.float32),
            pltpu.SemaphoreType.DMA,
        ],
    )


def _g1_body(x_ref, w_ref, dinv_ref, o_ref):
    hm = jnp.dot(x_ref[...], w_ref[...], preferred_element_type=jnp.float32)
    o_ref[...] = hm * dinv_ref[...]


def _y1g2_body(agg_ref, g1_ref, dinv_ref, b1_ref, w2_ref, o_ref):
    dv = dinv_ref[...]
    y1 = jnp.maximum(dv * (agg_ref[...] + g1_ref[...]) + b1_ref[...], 0.0)
    o_ref[...] = jnp.dot(y1, w2_ref[...],
                         preferred_element_type=jnp.float32) * dv


def _make_pool(r, h, gsplit):
    def body(agg_ref, g2_ref, dinv_ref, b2_ref, batch_ref, pooled_ref,
             cnt_ref):
        i = pl.program_id(0)
        z = dinv_ref[...] * (agg_ref[...] + g2_ref[...]) + b2_ref[...]
        seg = batch_ref[...] + jnp.where(i >= gsplit, G, 0)
        oh = (seg == lax.broadcasted_iota(jnp.int32, (1, 2 * G), 1)
              ).astype(jnp.float32)
        ps = lax.dot_general(oh, z, (((0,), (0,)), ((), ())),
                             preferred_element_type=jnp.float32)
        cs = lax.dot_general(oh, jnp.ones((r, 8), jnp.float32),
                             (((0,), (0,)), ((), ())),
                             preferred_element_type=jnp.float32)

        @pl.when(i == 0)
        def _():
            pooled_ref[...] = ps
            cnt_ref[...] = cs

        @pl.when(i > 0)
        def _():
            pooled_ref[...] = pooled_ref[...] + ps
            cnt_ref[...] = cnt_ref[...] + cs

    return body


def _mlp_body(pooled_ref, cnt_ref, wl1_ref, bl1_ref, wl2_ref, bl2_ref, o_ref):
    cnt = jnp.maximum(cnt_ref[...][:, 0:1], 1.0)
    emb = pooled_ref[...] / cnt
    comb = jnp.concatenate([emb[0:G], emb[G:2 * G]], axis=1)
    hm = jnp.maximum(
        jnp.dot(comb, wl1_ref[...], preferred_element_type=jnp.float32)
        + bl1_ref[...], 0.0)
    o_ref[...] = (jnp.dot(hm, wl2_ref[...], preferred_element_type=jnp.float32)
                  + bl2_ref[...])


def kernel(x1, edge_index1, batch1, x2, edge_index2, batch2,
           Wc1, bc1, Wc2, bc2, Wl1, bl1, Wl2, bl2):
    n = x1.shape[0]
    e = edge_index1.shape[1]
    h = Wc1.shape[1]
    n2 = 2 * n
    npass = -(-n2 // (NC * C_CH))
    nrows_out = npass * NC * C_CH
    padv = nrows_out
    hsz = padv + 16
    epad = _ceil_to(e, NC * NS * SA)

    padlen = epad - e
    padsrc = (jnp.arange(padlen, dtype=jnp.int32) * 37) % n
    paddst = jnp.full((padlen,), padv, jnp.int32)
    srcall = jnp.concatenate(
        [edge_index1[0], padsrc, edge_index2[0] + n, padsrc + n])
    dstall = jnp.concatenate(
        [edge_index1[1], paddst, edge_index2[1] + n, paddst])
    xcat = jnp.pad(jnp.concatenate([x1, x2], 0), ((0, 0), (0, 2)))
    wc1p = jnp.pad(Wc1, ((0, 2), (0, 0)))
    batchcat = jnp.concatenate([batch1, batch2]).reshape(n2, 1)
    bc1r = bc1.reshape(1, h)
    bc2r = bc2.reshape(1, h)
    bl1r = bl1.reshape(1, h)
    bl2r = bl2.reshape(1, 1)

    part = _make_deg(2 * epad, hsz)(dstall)
    dinv = _make_dinv(hsz, nrows_out)(part).reshape(nrows_out, 1)

    r = 1000
    nblk = n2 // r
    g1 = pl.pallas_call(
        _g1_body,
        grid=(nblk,),
        in_specs=[
            pl.BlockSpec((r, 8), lambda i: (i, 0)),
            pl.BlockSpec((8, h), lambda i: (0, 0)),
            pl.BlockSpec((r, 1), lambda i: (i, 0)),
        ],
        out_specs=pl.BlockSpec((r, h), lambda i: (i, 0)),
        out_shape=jax.ShapeDtypeStruct((n2, h), jnp.float32),
    )(xcat, wc1p, dinv)

    agg_fn = _make_agg(2 * epad, h, npass, nrows_out)
    agg1 = agg_fn(g1, srcall, dstall)

    g2 = pl.pallas_call(
        _y1g2_body,
        grid=(nblk,),
        in_specs=[
            pl.BlockSpec((r, h), lambda i: (i, 0)),
            pl.BlockSpec((r, h), lambda i: (i, 0)),
            pl.BlockSpec((r, 1), lambda i: (i, 0)),
            pl.BlockSpec((1, h), lambda i: (0, 0)),
            pl.BlockSpec((h, h), lambda i: (0, 0)),
        ],
        out_specs=pl.BlockSpec((r, h), lambda i: (i, 0)),
        out_shape=jax.ShapeDtypeStruct((n2, h), jnp.float32),
    )(agg1, g1, dinv, bc1r, Wc2)

    agg2 = agg_fn(g2, srcall, dstall)

    pooled, cnt = pl.pallas_call(
        _make_pool(r, h, n // r),
        grid=(nblk,),
        in_specs=[
            pl.BlockSpec((r, h), lambda i: (i, 0)),
            pl.BlockSpec((r, h), lambda i: (i, 0)),
            pl.BlockSpec((r, 1), lambda i: (i, 0)),
            pl.BlockSpec((1, h), lambda i: (0, 0)),
            pl.BlockSpec((r, 1), lambda i: (i, 0)),
        ],
        out_specs=[
            pl.BlockSpec((2 * G, h), lambda i: (0, 0)),
            pl.BlockSpec((2 * G, 8), lambda i: (0, 0)),
        ],
        out_shape=[
            jax.ShapeDtypeStruct((2 * G, h), jnp.float32),
            jax.ShapeDtypeStruct((2 * G, 8), jnp.float32),
        ],
    )(agg2, g2, dinv, bc2r, batchcat)

    out = pl.pallas_call(
        _mlp_body,
        grid=(1,),
        in_specs=[
            pl.BlockSpec((2 * G, h), lambda i: (0, 0)),
            pl.BlockSpec((2 * G, 8), lambda i: (0, 0)),
            pl.BlockSpec((2 * h, h), lambda i: (0, 0)),
            pl.BlockSpec((1, h), lambda i: (0, 0)),
            pl.BlockSpec((h, 1), lambda i: (0, 0)),
            pl.BlockSpec((1, 1), lambda i: (0, 0)),
        ],
        out_specs=pl.BlockSpec((G, 1), lambda i: (0, 0)),
        out_shape=jax.ShapeDtypeStruct((G, 1), jnp.float32),
    )(pooled, cnt, Wl1, bl1r, Wl2, bl2r)
    return out
